# Initial kernel scaffold; baseline (speedup 1.0000x reference)
import jax
import jax.numpy as jnp
from jax.experimental import pallas as pl


def kernel(x, edge_index, W1, b1, W2, b2):
    raise NotImplementedError("write your pallas kernel here")


# speedup vs baseline: 8.3038x; 8.3038x over previous
import functools

import jax
import jax.numpy as jnp
from jax import lax
from jax.experimental import pallas as pl
from jax.experimental.pallas import tpu as pltpu
from jax.experimental.pallas import tpu_sc as plsc

NC = 2
NS = 16
NW = NC * NS
DEG_W = 16
_SC_PARAMS = pltpu.CompilerParams(use_tc_tiling_on_sc=False)


def _make_deg_kernel(n_pad, chunks, k):
    rows_per_tile = n_pad // NS
    mesh = plsc.VectorSubcoreMesh(core_axis_name="c", subcore_axis_name="s")

    @functools.partial(
        pl.kernel,
        out_type=(
            jax.ShapeDtypeStruct((NC, n_pad, DEG_W), jnp.float32),
            jax.ShapeDtypeStruct((NC, n_pad, DEG_W), jnp.float32),
        ),
        mesh=mesh,
        scratch_types=[
            pltpu.VMEM_SHARED((n_pad, DEG_W), jnp.float32),
            pltpu.VMEM_SHARED((n_pad, DEG_W), jnp.float32),
            pltpu.VMEM((chunks, k), jnp.int32),
            pltpu.VMEM((chunks, k), jnp.int32),
            pltpu.VMEM((k, DEG_W), jnp.float32),
        ],
        compiler_params=_SC_PARAMS,
    )
    def deg(srcr_hbm, dstr_hbm, ones_hbm, zrows_hbm, outs_hbm, outd_hbm,
            accs, accd, sidx, didx, ones_v):
        ci = lax.axis_index("c")
        si = lax.axis_index("s")
        wid = ci * NS + si
        r0 = si * rows_per_tile
        pltpu.sync_copy(zrows_hbm, accs.at[pl.ds(r0, rows_per_tile)])
        pltpu.sync_copy(zrows_hbm, accd.at[pl.ds(r0, rows_per_tile)])
        pltpu.sync_copy(ones_hbm, ones_v)
        pltpu.sync_copy(srcr_hbm.at[wid], sidx)
        pltpu.sync_copy(dstr_hbm.at[wid], didx)
        plsc.subcore_barrier()

        def body(c, carry):
            pltpu.sync_copy(ones_v, accs.at[sidx.at[c]], add=True)
            pltpu.sync_copy(ones_v, accd.at[didx.at[c]], add=True)
            return carry

        lax.fori_loop(0, chunks, body, 0, unroll=False)
        plsc.subcore_barrier()
        pltpu.sync_copy(accs.at[pl.ds(r0, rows_per_tile)],
                        outs_hbm.at[ci, pl.ds(r0, rows_per_tile)])
        pltpu.sync_copy(accd.at[pl.ds(r0, rows_per_tile)],
                        outd_hbm.at[ci, pl.ds(r0, rows_per_tile)])

    return deg


def _make_agg_kernel(n_pad, feat, chunks, k):
    rows_per_tile = n_pad // NS
    mesh = plsc.VectorSubcoreMesh(core_axis_name="c", subcore_axis_name="s")

    @functools.partial(
        pl.kernel,
        out_type=jax.ShapeDtypeStruct((NC, n_pad, feat), jnp.float32),
        mesh=mesh,
        scratch_types=[
            pltpu.VMEM_SHARED((n_pad, feat), jnp.float32),
            pltpu.VMEM((chunks, k), jnp.int32),
            pltpu.VMEM((chunks, k), jnp.int32),
            pltpu.VMEM((k, feat), jnp.float32),
            pltpu.SemaphoreType.DMA,
        ],
        compiler_params=_SC_PARAMS,
    )
    def agg(h_hbm, srcr_hbm, dstr_hbm, zrows_hbm, out_hbm,
            acc, sidx, didx, rows, sem):
        ci = lax.axis_index("c")
        si = lax.axis_index("s")
        wid = ci * NS + si
        r0 = si * rows_per_tile
        pltpu.sync_copy(zrows_hbm, acc.at[pl.ds(r0, rows_per_tile)])
        pltpu.sync_copy(srcr_hbm.at[wid], sidx)
        pltpu.sync_copy(dstr_hbm.at[wid], didx)
        plsc.subcore_barrier()

        def body(c, carry):
            pltpu.async_copy(h_hbm.at[sidx.at[c]], rows, sem).wait()
            pltpu.sync_copy(rows, acc.at[didx.at[c]], add=True)
            return carry

        lax.fori_loop(0, chunks, body, 0, unroll=False)
        plsc.subcore_barrier()
        pltpu.sync_copy(acc.at[pl.ds(r0, rows_per_tile)],
                        out_hbm.at[ci, pl.ds(r0, rows_per_tile)])

    return agg


def _tc_scale_matmul(x, w, degs, blk=400):
    n, f = x.shape
    h = w.shape[1]

    def body(x_ref, w_ref, d_ref, o_ref):
        deg = d_ref[0, :, 0:1] + d_ref[1, :, 0:1]
        nrm = lax.rsqrt(jnp.maximum(deg, 1.0))
        o_ref[...] = jnp.dot(x_ref[...], w_ref[...],
                             preferred_element_type=jnp.float32) * nrm

    return pl.pallas_call(
        body,
        grid=(n // blk,),
        in_specs=[
            pl.BlockSpec((blk, f), lambda i: (i, 0)),
            pl.BlockSpec((f, h), lambda i: (0, 0)),
            pl.BlockSpec((2, blk, DEG_W), lambda i: (0, i, 0)),
        ],
        out_specs=pl.BlockSpec((blk, h), lambda i: (i, 0)),
        out_shape=jax.ShapeDtypeStruct((n, h), jnp.float32),
    )(x, w, degs)


def _tc_mid(n, agg1, degs, degd, b1, w2p, blk=400):
    f = agg1.shape[2]
    c2 = w2p.shape[1]

    def body(a_ref, ds_ref, dd_ref, b_ref, w_ref, o_ref):
        nin = lax.rsqrt(jnp.maximum(dd_ref[0, :, 0:1] + dd_ref[1, :, 0:1], 1.0))
        a = a_ref[0] + a_ref[1]
        z = jnp.maximum(a * nin + b_ref[...], 0.0)
        nout = lax.rsqrt(jnp.maximum(ds_ref[0, :, 0:1] + ds_ref[1, :, 0:1], 1.0))
        o_ref[...] = jnp.dot(z, w_ref[...],
                             preferred_element_type=jnp.float32) * nout

    return pl.pallas_call(
        body,
        grid=(n // blk,),
        in_specs=[
            pl.BlockSpec((2, blk, f), lambda i: (0, i, 0)),
            pl.BlockSpec((2, blk, DEG_W), lambda i: (0, i, 0)),
            pl.BlockSpec((2, blk, DEG_W), lambda i: (0, i, 0)),
            pl.BlockSpec((1, f), lambda i: (0, 0)),
            pl.BlockSpec((f, c2), lambda i: (0, 0)),
        ],
        out_specs=pl.BlockSpec((blk, c2), lambda i: (i, 0)),
        out_shape=jax.ShapeDtypeStruct((n, c2), jnp.float32),
    )(agg1, degs, degd, b1, w2p)


def _tc_final(n, agg2, degd, b2p, blk=400):
    c2 = agg2.shape[2]

    def body(a_ref, dd_ref, b_ref, o_ref):
        nin = lax.rsqrt(jnp.maximum(dd_ref[0, :, 0:1] + dd_ref[1, :, 0:1], 1.0))
        o_ref[...] = (a_ref[0] + a_ref[1]) * nin + b_ref[...]

    return pl.pallas_call(
        body,
        grid=(n // blk,),
        in_specs=[
            pl.BlockSpec((2, blk, c2), lambda i: (0, i, 0)),
            pl.BlockSpec((2, blk, DEG_W), lambda i: (0, i, 0)),
            pl.BlockSpec((1, c2), lambda i: (0, 0)),
        ],
        out_specs=pl.BlockSpec((blk, c2), lambda i: (i, 0)),
        out_shape=jax.ShapeDtypeStruct((n, c2), jnp.float32),
    )(agg2, degd, b2p)


def kernel(x, edge_index, W1, b1, W2, b2):
    n, f_in = x.shape
    h_feats = W1.shape[1]
    n_cls = W2.shape[1]
    e = edge_index.shape[1]

    k = 80
    assert e % (NW * k) == 0
    chunks = e // (NW * k)
    n_pad = ((n + NS * 8 - 1) // (NS * 8)) * (NS * 8)
    rows_per_tile = n_pad // NS
    c_pad = 32

    srcr = edge_index[0].reshape(NW, chunks, k)
    dstr = edge_index[1].reshape(NW, chunks, k)
    ones16 = jnp.ones((k, DEG_W), jnp.float32)
    z16 = jnp.zeros((rows_per_tile, DEG_W), jnp.float32)
    z128 = jnp.zeros((rows_per_tile, h_feats), jnp.float32)
    z32 = jnp.zeros((rows_per_tile, c_pad), jnp.float32)
    w2p = jnp.pad(W2, ((0, 0), (0, c_pad - n_cls)))
    b1r = b1.reshape(1, h_feats)
    b2p = jnp.pad(b2, (0, c_pad - n_cls)).reshape(1, c_pad)

    degs, degd = _make_deg_kernel(n_pad, chunks, k)(srcr, dstr, ones16, z16)
    h1 = _tc_scale_matmul(x, W1, degs)
    p1 = _make_agg_kernel(n_pad, h_feats, chunks, k)(h1, srcr, dstr, z128)
    h2 = _tc_mid(n, p1, degs, degd, b1r, w2p)
    p2 = _make_agg_kernel(n_pad, c_pad, chunks, k)(h2, srcr, dstr, z32)
    out = _tc_final(n, p2, degd, b2p)
    return out[:, :n_cls]


# speedup vs baseline: 12.0834x; 1.4552x over previous
import functools

import jax
import jax.numpy as jnp
from jax import lax
from jax.experimental import pallas as pl
from jax.experimental.pallas import tpu as pltpu
from jax.experimental.pallas import tpu_sc as plsc

NC = 2
NS = 16
NW = NC * NS
DEG_W = 16
_SC_PARAMS = pltpu.CompilerParams(use_tc_tiling_on_sc=False)


def _make_deg_kernel(n_pad, chunks, k):
    rows_per_tile = n_pad // NS
    mesh = plsc.VectorSubcoreMesh(core_axis_name="c", subcore_axis_name="s")

    @functools.partial(
        pl.kernel,
        out_type=(
            jax.ShapeDtypeStruct((NC, n_pad, DEG_W), jnp.float32),
            jax.ShapeDtypeStruct((NC, n_pad, DEG_W), jnp.float32),
        ),
        mesh=mesh,
        scratch_types=[
            pltpu.VMEM_SHARED((n_pad, DEG_W), jnp.float32),
            pltpu.VMEM_SHARED((n_pad, DEG_W), jnp.float32),
            pltpu.VMEM((chunks, k), jnp.int32),
            pltpu.VMEM((chunks, k), jnp.int32),
            pltpu.VMEM((k, DEG_W), jnp.float32),
            pltpu.SemaphoreType.DMA,
        ],
        compiler_params=_SC_PARAMS,
    )
    def deg(srcr_hbm, dstr_hbm, ones_hbm, zrows_hbm, outs_hbm, outd_hbm,
            accs, accd, sidx, didx, ones_v, sem):
        ci = lax.axis_index("c")
        si = lax.axis_index("s")
        wid = ci * NS + si
        r0 = si * rows_per_tile
        pltpu.sync_copy(zrows_hbm, accs.at[pl.ds(r0, rows_per_tile)])
        pltpu.sync_copy(zrows_hbm, accd.at[pl.ds(r0, rows_per_tile)])
        pltpu.sync_copy(ones_hbm, ones_v)
        pltpu.sync_copy(srcr_hbm.at[wid], sidx)
        pltpu.sync_copy(dstr_hbm.at[wid], didx)
        plsc.subcore_barrier()

        def body(c2, carry):
            c = 2 * c2
            d0 = pltpu.async_copy(ones_v, accs.at[sidx.at[c]], sem, add=True)
            d1 = pltpu.async_copy(ones_v, accd.at[didx.at[c]], sem, add=True)
            d2 = pltpu.async_copy(ones_v, accs.at[sidx.at[c + 1]], sem,
                                  add=True)
            d3 = pltpu.async_copy(ones_v, accd.at[didx.at[c + 1]], sem,
                                  add=True)
            d0.wait()
            d1.wait()
            d2.wait()
            d3.wait()
            return carry

        assert chunks % 2 == 1
        lax.fori_loop(0, chunks // 2, body, 0, unroll=False)
        dl0 = pltpu.async_copy(ones_v, accs.at[sidx.at[chunks - 1]], sem,
                               add=True)
        dl1 = pltpu.async_copy(ones_v, accd.at[didx.at[chunks - 1]], sem,
                               add=True)
        dl0.wait()
        dl1.wait()
        plsc.subcore_barrier()
        pltpu.sync_copy(accs.at[pl.ds(r0, rows_per_tile)],
                        outs_hbm.at[ci, pl.ds(r0, rows_per_tile)])
        pltpu.sync_copy(accd.at[pl.ds(r0, rows_per_tile)],
                        outd_hbm.at[ci, pl.ds(r0, rows_per_tile)])

    return deg


def _make_agg_kernel(n_pad, feat, chunks, k):
    rows_per_tile = n_pad // NS
    mesh = plsc.VectorSubcoreMesh(core_axis_name="c", subcore_axis_name="s")

    @functools.partial(
        pl.kernel,
        out_type=jax.ShapeDtypeStruct((NC, n_pad, feat), jnp.float32),
        mesh=mesh,
        scratch_types=[
            pltpu.VMEM_SHARED((n_pad, feat), jnp.float32),
            pltpu.VMEM((chunks, k), jnp.int32),
            pltpu.VMEM((chunks, k), jnp.int32),
            pltpu.VMEM((k, feat), jnp.float32),
            pltpu.VMEM((k, feat), jnp.float32),
            pltpu.SemaphoreType.DMA,
            pltpu.SemaphoreType.DMA,
        ],
        compiler_params=_SC_PARAMS,
    )
    def agg(h_hbm, srcr_hbm, dstr_hbm, zrows_hbm, out_hbm,
            acc, sidx, didx, rows0, rows1, sem0, sem1):
        ci = lax.axis_index("c")
        si = lax.axis_index("s")
        wid = ci * NS + si
        r0 = si * rows_per_tile
        pltpu.sync_copy(zrows_hbm, acc.at[pl.ds(r0, rows_per_tile)])
        pltpu.sync_copy(srcr_hbm.at[wid], sidx)
        pltpu.sync_copy(dstr_hbm.at[wid], didx)
        plsc.subcore_barrier()

        assert chunks % 2 == 1 and chunks >= 3
        pltpu.async_copy(h_hbm.at[sidx.at[0]], rows0, sem0)
        pltpu.async_copy(h_hbm.at[sidx.at[1]], rows1, sem1)

        def body(c2, carry):
            c = 2 * c2
            pltpu.make_async_copy(h_hbm.at[sidx.at[c]], rows0, sem0).wait()
            pltpu.sync_copy(rows0, acc.at[didx.at[c]], add=True)
            pltpu.async_copy(h_hbm.at[sidx.at[c + 2]], rows0, sem0)
            pltpu.make_async_copy(h_hbm.at[sidx.at[c + 1]], rows1,
                                  sem1).wait()
            pltpu.sync_copy(rows1, acc.at[didx.at[c + 1]], add=True)

            @pl.when(c2 < chunks // 2 - 1)
            def _():
                pltpu.async_copy(h_hbm.at[sidx.at[c + 3]], rows1, sem1)

            return carry

        lax.fori_loop(0, chunks // 2, body, 0, unroll=False)
        pltpu.make_async_copy(h_hbm.at[sidx.at[chunks - 1]], rows0,
                              sem0).wait()
        pltpu.sync_copy(rows0, acc.at[didx.at[chunks - 1]], add=True)
        plsc.subcore_barrier()
        pltpu.sync_copy(acc.at[pl.ds(r0, rows_per_tile)],
                        out_hbm.at[ci, pl.ds(r0, rows_per_tile)])

    return agg


def _tc_scale_matmul(x, w, degs, blk=400):
    n, f = x.shape
    h = w.shape[1]

    def body(x_ref, w_ref, d_ref, o_ref):
        deg = d_ref[0, :, 0:1] + d_ref[1, :, 0:1]
        nrm = lax.rsqrt(jnp.maximum(deg, 1.0))
        o_ref[...] = jnp.dot(x_ref[...], w_ref[...],
                             preferred_element_type=jnp.float32) * nrm

    return pl.pallas_call(
        body,
        grid=(n // blk,),
        in_specs=[
            pl.BlockSpec((blk, f), lambda i: (i, 0)),
            pl.BlockSpec((f, h), lambda i: (0, 0)),
            pl.BlockSpec((2, blk, DEG_W), lambda i: (0, i, 0)),
        ],
        out_specs=pl.BlockSpec((blk, h), lambda i: (i, 0)),
        out_shape=jax.ShapeDtypeStruct((n, h), jnp.float32),
    )(x, w, degs)


def _tc_mid(n, agg1, degs, degd, b1, w2p, blk=400):
    f = agg1.shape[2]
    c2 = w2p.shape[1]

    def body(a_ref, ds_ref, dd_ref, b_ref, w_ref, o_ref):
        nin = lax.rsqrt(jnp.maximum(dd_ref[0, :, 0:1] + dd_ref[1, :, 0:1], 1.0))
        a = a_ref[0] + a_ref[1]
        z = jnp.maximum(a * nin + b_ref[...], 0.0)
        nout = lax.rsqrt(jnp.maximum(ds_ref[0, :, 0:1] + ds_ref[1, :, 0:1], 1.0))
        o_ref[...] = jnp.dot(z, w_ref[...],
                             preferred_element_type=jnp.float32) * nout

    return pl.pallas_call(
        body,
        grid=(n // blk,),
        in_specs=[
            pl.BlockSpec((2, blk, f), lambda i: (0, i, 0)),
            pl.BlockSpec((2, blk, DEG_W), lambda i: (0, i, 0)),
            pl.BlockSpec((2, blk, DEG_W), lambda i: (0, i, 0)),
            pl.BlockSpec((1, f), lambda i: (0, 0)),
            pl.BlockSpec((f, c2), lambda i: (0, 0)),
        ],
        out_specs=pl.BlockSpec((blk, c2), lambda i: (i, 0)),
        out_shape=jax.ShapeDtypeStruct((n, c2), jnp.float32),
    )(agg1, degs, degd, b1, w2p)


def _tc_final(n, agg2, degd, b2p, blk=400):
    c2 = agg2.shape[2]

    def body(a_ref, dd_ref, b_ref, o_ref):
        nin = lax.rsqrt(jnp.maximum(dd_ref[0, :, 0:1] + dd_ref[1, :, 0:1], 1.0))
        o_ref[...] = (a_ref[0] + a_ref[1]) * nin + b_ref[...]

    return pl.pallas_call(
        body,
        grid=(n // blk,),
        in_specs=[
            pl.BlockSpec((2, blk, c2), lambda i: (0, i, 0)),
            pl.BlockSpec((2, blk, DEG_W), lambda i: (0, i, 0)),
            pl.BlockSpec((1, c2), lambda i: (0, 0)),
        ],
        out_specs=pl.BlockSpec((blk, c2), lambda i: (i, 0)),
        out_shape=jax.ShapeDtypeStruct((n, c2), jnp.float32),
    )(agg2, degd, b2p)


def kernel(x, edge_index, W1, b1, W2, b2):
    n, f_in = x.shape
    h_feats = W1.shape[1]
    n_cls = W2.shape[1]
    e = edge_index.shape[1]

    k = 80
    assert e % (NW * k) == 0
    chunks = e // (NW * k)
    n_pad = ((n + NS * 8 - 1) // (NS * 8)) * (NS * 8)
    rows_per_tile = n_pad // NS
    c_pad = 32

    srcr = edge_index[0].reshape(NW, chunks, k)
    dstr = edge_index[1].reshape(NW, chunks, k)
    ones16 = jnp.ones((k, DEG_W), jnp.float32)
    z16 = jnp.zeros((rows_per_tile, DEG_W), jnp.float32)
    z128 = jnp.zeros((rows_per_tile, h_feats), jnp.float32)
    z32 = jnp.zeros((rows_per_tile, c_pad), jnp.float32)
    w2p = jnp.pad(W2, ((0, 0), (0, c_pad - n_cls)))
    b1r = b1.reshape(1, h_feats)
    b2p = jnp.pad(b2, (0, c_pad - n_cls)).reshape(1, c_pad)

    degs, degd = _make_deg_kernel(n_pad, chunks, k)(srcr, dstr, ones16, z16)
    h1 = _tc_scale_matmul(x, W1, degs)
    p1 = _make_agg_kernel(n_pad, h_feats, chunks, k)(h1, srcr, dstr, z128)
    h2 = _tc_mid(n, p1, degs, degd, b1r, w2p)
    p2 = _make_agg_kernel(n_pad, c_pad, chunks, k)(h2, srcr, dstr, z32)
    out = _tc_final(n, p2, degd, b2p)
    return out[:, :n_cls]


# speedup vs baseline: 13.1164x; 1.0855x over previous
import functools

import jax
import jax.numpy as jnp
from jax import lax
from jax.experimental import pallas as pl
from jax.experimental.pallas import tpu as pltpu
from jax.experimental.pallas import tpu_sc as plsc

NC = 2
NS = 16
NW = NC * NS
DEG_W = 16
K_E = 128
_SC_PARAMS = pltpu.CompilerParams(use_tc_tiling_on_sc=False)


def _fill_rows(buf, rows, width, value):
    vec = jnp.full((16,), value, jnp.float32)
    for r in range(rows):
        for j in range(width // 16):
            buf[r, 16 * j:16 * (j + 1)] = vec


def _make_deg_kernel(n_pad, chunks):
    rows_per_tile = n_pad // NS
    mesh = plsc.VectorSubcoreMesh(core_axis_name="c", subcore_axis_name="s")

    @functools.partial(
        pl.kernel,
        out_type=(
            jax.ShapeDtypeStruct((NC, n_pad, DEG_W), jnp.float32),
            jax.ShapeDtypeStruct((NC, n_pad, DEG_W), jnp.float32),
        ),
        mesh=mesh,
        scratch_types=[
            pltpu.VMEM_SHARED((n_pad, DEG_W), jnp.float32),
            pltpu.VMEM_SHARED((n_pad, DEG_W), jnp.float32),
            pltpu.VMEM((chunks, K_E), jnp.int32),
            pltpu.VMEM((chunks, K_E), jnp.int32),
            pltpu.VMEM((K_E, DEG_W), jnp.float32),
            pltpu.SemaphoreType.DMA,
        ],
        compiler_params=_SC_PARAMS,
    )
    def deg(srcr_hbm, dstr_hbm, outs_hbm, outd_hbm,
            accs, accd, sidx, didx, ones_v, sem):
        ci = lax.axis_index("c")
        si = lax.axis_index("s")
        wid = ci * NS + si
        r0 = si * rows_per_tile
        _fill_rows(ones_v, K_E, DEG_W, 0.0)
        for b in range(rows_per_tile // K_E):
            pltpu.sync_copy(ones_v, accs.at[pl.ds(r0 + b * K_E, K_E)])
            pltpu.sync_copy(ones_v, accd.at[pl.ds(r0 + b * K_E, K_E)])
        _fill_rows(ones_v, K_E, DEG_W, 1.0)
        pltpu.sync_copy(srcr_hbm.at[wid], sidx)
        pltpu.sync_copy(dstr_hbm.at[wid], didx)
        plsc.subcore_barrier()

        def body(c2, carry):
            c = 2 * c2
            d0 = pltpu.async_copy(ones_v, accs.at[sidx.at[c]], sem, add=True)
            d1 = pltpu.async_copy(ones_v, accd.at[didx.at[c]], sem, add=True)
            d2 = pltpu.async_copy(ones_v, accs.at[sidx.at[c + 1]], sem,
                                  add=True)
            d3 = pltpu.async_copy(ones_v, accd.at[didx.at[c + 1]], sem,
                                  add=True)
            d0.wait()
            d1.wait()
            d2.wait()
            d3.wait()
            return carry

        assert chunks % 2 == 0
        lax.fori_loop(0, chunks // 2, body, 0, unroll=False)
        plsc.subcore_barrier()
        pltpu.sync_copy(accs.at[pl.ds(r0, rows_per_tile)],
                        outs_hbm.at[ci, pl.ds(r0, rows_per_tile)])
        pltpu.sync_copy(accd.at[pl.ds(r0, rows_per_tile)],
                        outd_hbm.at[ci, pl.ds(r0, rows_per_tile)])

    return deg


def _make_agg_kernel(n_pad, feat, chunks, phases):
    rows_per_tile = n_pad // NS
    assert chunks % phases == 0
    pchunks = chunks // phases
    assert pchunks % 2 == 0 and pchunks >= 4
    mesh = plsc.VectorSubcoreMesh(core_axis_name="c", subcore_axis_name="s")
    from_hbm_zeros = feat % 128 == 0

    zsrc = ([pltpu.VMEM((K_E, feat), jnp.float32)]
            if not from_hbm_zeros else [])

    @functools.partial(
        pl.kernel,
        out_type=jax.ShapeDtypeStruct((NC, n_pad, feat), jnp.float32),
        mesh=mesh,
        scratch_types=[
            pltpu.VMEM_SHARED((n_pad, feat), jnp.float32),
            pltpu.VMEM((pchunks, K_E), jnp.int32),
            pltpu.VMEM((pchunks, K_E), jnp.int32),
            pltpu.VMEM((K_E, feat), jnp.float32),
            pltpu.VMEM((K_E, feat), jnp.float32),
            pltpu.SemaphoreType.DMA,
            pltpu.SemaphoreType.DMA,
        ] + zsrc,
        compiler_params=_SC_PARAMS,
    )
    def agg(h_hbm, srcr_hbm, dstr_hbm, *rest):
        if from_hbm_zeros:
            zrows_hbm, out_hbm, acc, sidx, didx, rows0, rows1, sem0, sem1 = rest
        else:
            out_hbm, acc, sidx, didx, rows0, rows1, sem0, sem1, zbuf = rest
        ci = lax.axis_index("c")
        si = lax.axis_index("s")
        wid = ci * NS + si
        r0 = si * rows_per_tile
        if from_hbm_zeros:
            pltpu.sync_copy(zrows_hbm, acc.at[pl.ds(r0, rows_per_tile)])
        else:
            _fill_rows(zbuf, K_E, feat, 0.0)
            for b in range(rows_per_tile // K_E):
                pltpu.sync_copy(zbuf, acc.at[pl.ds(r0 + b * K_E, K_E)])
        plsc.subcore_barrier()

        for phase in range(phases):
            pltpu.sync_copy(srcr_hbm.at[wid, pl.ds(phase * pchunks, pchunks)],
                            sidx)
            pltpu.sync_copy(dstr_hbm.at[wid, pl.ds(phase * pchunks, pchunks)],
                            didx)
            pltpu.async_copy(h_hbm.at[sidx.at[0]], rows0, sem0)
            pltpu.async_copy(h_hbm.at[sidx.at[1]], rows1, sem1)

            def body(c2, carry):
                c = 2 * c2
                pltpu.make_async_copy(h_hbm.at[sidx.at[c]], rows0,
                                      sem0).wait()
                pltpu.sync_copy(rows0, acc.at[didx.at[c]], add=True)
                pltpu.async_copy(h_hbm.at[sidx.at[c + 2]], rows0, sem0)
                pltpu.make_async_copy(h_hbm.at[sidx.at[c + 1]], rows1,
                                      sem1).wait()
                pltpu.sync_copy(rows1, acc.at[didx.at[c + 1]], add=True)
                pltpu.async_copy(h_hbm.at[sidx.at[c + 3]], rows1, sem1)
                return carry

            lax.fori_loop(0, pchunks // 2 - 1, body, 0, unroll=False)
            pltpu.make_async_copy(h_hbm.at[sidx.at[pchunks - 2]], rows0,
                                  sem0).wait()
            pltpu.sync_copy(rows0, acc.at[didx.at[pchunks - 2]], add=True)
            pltpu.make_async_copy(h_hbm.at[sidx.at[pchunks - 1]], rows1,
                                  sem1).wait()
            pltpu.sync_copy(rows1, acc.at[didx.at[pchunks - 1]], add=True)

        plsc.subcore_barrier()
        pltpu.sync_copy(acc.at[pl.ds(r0, rows_per_tile)],
                        out_hbm.at[ci, pl.ds(r0, rows_per_tile)])

    return agg


def _tc_matmul(x, w, out_cols, blk=512):
    n, f = x.shape
    h = w.shape[1]

    def body(x_ref, w_ref, o_ref):
        o_ref[...] = jnp.dot(x_ref[...], w_ref[...][:, :out_cols],
                             preferred_element_type=jnp.float32)

    return pl.pallas_call(
        body,
        grid=(n // blk,),
        in_specs=[
            pl.BlockSpec((blk, f), lambda i: (i, 0)),
            pl.BlockSpec((f, h), lambda i: (0, 0)),
        ],
        out_specs=pl.BlockSpec((blk, out_cols), lambda i: (i, 0)),
        out_shape=jax.ShapeDtypeStruct((n, out_cols), jnp.float32),
    )(x, w)


def kernel(x, edge_index, W1, b1, W2, b2):
    n, f_in = x.shape
    h_feats = W1.shape[1]
    n_cls = W2.shape[1]
    e = edge_index.shape[1]

    assert e % NW == 0
    epw = e // NW
    n_pad = ((n + NS * K_E - 1) // (NS * K_E)) * (NS * K_E)
    rows_per_tile = n_pad // NS
    c_pad = 32
    chunks = 2 * (-(-epw // (2 * K_E)))
    npad_e = chunks * K_E - epw

    pad_idx = n + (jnp.arange(npad_e, dtype=jnp.int32) % (n_pad - n))
    srcr = jnp.concatenate(
        [edge_index[0].reshape(NW, epw),
         jnp.broadcast_to(pad_idx, (NW, npad_e))], axis=1
    ).reshape(NW, chunks, K_E)
    dstr = jnp.concatenate(
        [edge_index[1].reshape(NW, epw),
         jnp.broadcast_to(pad_idx, (NW, npad_e))], axis=1
    ).reshape(NW, chunks, K_E)

    z128 = jnp.zeros((rows_per_tile, h_feats), jnp.float32)
    w2f = jnp.pad(W2, ((0, 0), (0, h_feats - n_cls)))
    xp = jnp.pad(x, ((0, n_pad - n), (0, 0)))

    degs, degd = _make_deg_kernel(n_pad, chunks)(srcr, dstr)
    nout = lax.rsqrt(jnp.clip(degs[0, :, 0:1] + degs[1, :, 0:1], 1.0))
    nin = lax.rsqrt(jnp.clip(degd[0, :, 0:1] + degd[1, :, 0:1], 1.0))

    h1 = _tc_matmul(xp * nout, W1, h_feats)
    p1 = _make_agg_kernel(n_pad, h_feats, chunks, 2)(h1, srcr, dstr, z128)
    z = jnp.maximum((p1[0] + p1[1]) * nin + b1.reshape(1, h_feats), 0.0)
    h2 = _tc_matmul(z * nout, w2f, c_pad)
    p2 = _make_agg_kernel(n_pad, c_pad, chunks, 1)(h2, srcr, dstr)
    out = (p2[0] + p2[1]) * nin + jnp.pad(b2, (0, c_pad - n_cls))
    return out[:n, :n_cls]
